# baseline (speedup 1.0000x reference)
import jax
import jax.numpy as jnp
from jax.experimental import pallas as pl
from jax.experimental.pallas import tpu as pltpu

_H = 768
_E = 64
_BLK = 256
_NB = _H // _BLK


def _router_body(x_ref, w1_ref, b1_ref, w2_ref, b2_ref, out_ref, h_ref):
    i = pl.program_id(0)
    x0 = x_ref[0]
    h = jax.lax.dot_general(
        x0, w1_ref[...], (((1,), (1,)), ((), ())),
        preferred_element_type=jnp.float32)
    h_ref[:, pl.ds(i * _BLK, _BLK)] = jnp.maximum(
        h + b1_ref[:, pl.ds(i * _BLK, _BLK)], 0.0)

    @pl.when(i == _NB - 1)
    def _():
        logits = jax.lax.dot_general(
            h_ref[...], w2_ref[...], (((1,), (1,)), ((), ())),
            preferred_element_type=jnp.float32) + b2_ref[...]
        ids = jax.lax.broadcasted_iota(jnp.int32, (1, _E), 1)
        v1 = jnp.max(logits, axis=1, keepdims=True)
        i1 = jnp.min(jnp.where(logits == v1, ids, _E), axis=1, keepdims=True)
        rest = jnp.where(ids == i1, -jnp.inf, logits)
        v2 = jnp.max(rest, axis=1, keepdims=True)
        i2 = jnp.min(jnp.where(rest == v2, ids, _E), axis=1, keepdims=True)
        e2 = jnp.exp(v2 - v1)
        denom = 1.0 + e2
        out_ref[...] = jnp.where(
            ids == i1, 1.0 / denom, jnp.where(ids == i2, e2 / denom, 0.0))


def kernel(x, W1, b1, W2, b2):
    out = pl.pallas_call(
        _router_body,
        grid=(_NB,),
        in_specs=[
            pl.BlockSpec((1, 1, _H), lambda i: (0, 0, 0)),
            pl.BlockSpec((_BLK, _H), lambda i: (i, 0)),
            pl.BlockSpec((1, _H), lambda i: (0, 0)),
            pl.BlockSpec((_E, _H), lambda i: (0, 0)),
            pl.BlockSpec((1, _E), lambda i: (0, 0)),
        ],
        out_specs=pl.BlockSpec((1, _E), lambda i: (0, 0)),
        out_shape=jax.ShapeDtypeStruct((1, _E), jnp.float32),
        scratch_shapes=[pltpu.VMEM((1, _H), jnp.float32)],
    )(x, W1, b1.reshape(1, _H), W2, b2.reshape(1, _E))
    return out.reshape(_E)


# speedup vs baseline: 1.1913x; 1.1913x over previous
import jax
import jax.numpy as jnp
from jax.experimental import pallas as pl

_H = 768
_E = 64


def _router_body(x_ref, w1_ref, b1_ref, w2_ref, b2_ref, out_ref):
    x0 = x_ref[0]
    h = jax.lax.dot_general(
        x0, w1_ref[...], (((1,), (1,)), ((), ())),
        preferred_element_type=jnp.float32)
    h = jnp.maximum(h + b1_ref[...], 0.0)
    logits = jax.lax.dot_general(
        h, w2_ref[...], (((1,), (1,)), ((), ())),
        preferred_element_type=jnp.float32)
    logits = logits + b2_ref[...]

    ids = jax.lax.broadcasted_iota(jnp.int32, (1, _E), 1)
    ninf = jnp.float32(-jnp.inf)
    v1 = jnp.max(logits, axis=1, keepdims=True)
    t1 = logits == v1
    i1 = jnp.min(jnp.where(t1, ids, _E), axis=1, keepdims=True)
    r2 = jnp.max(jnp.where(t1, ninf, logits), axis=1, keepdims=True)
    cnt = jnp.sum(t1.astype(jnp.float32), axis=1, keepdims=True)
    dup = cnt >= 2.0
    v2 = jnp.where(dup, v1, r2)
    i2a = jnp.min(jnp.where(t1 & (ids > i1), ids, _E), axis=1, keepdims=True)
    i2b = jnp.min(jnp.where(logits == r2, ids, _E), axis=1, keepdims=True)
    i2 = jnp.where(dup, i2a, i2b)

    e2 = jnp.exp(v2 - v1)
    denom = 1.0 + e2
    out_ref[...] = jnp.where(
        ids == i1, 1.0 / denom, jnp.where(ids == i2, e2 / denom, 0.0))


def kernel(x, W1, b1, W2, b2):
    out = pl.pallas_call(
        _router_body,
        grid=(1,),
        in_specs=[
            pl.BlockSpec((1, 1, _H), lambda i: (0, 0, 0)),
            pl.BlockSpec((_H, _H), lambda i: (0, 0)),
            pl.BlockSpec((1, _H), lambda i: (0, 0)),
            pl.BlockSpec((_E, _H), lambda i: (0, 0)),
            pl.BlockSpec((1, _E), lambda i: (0, 0)),
        ],
        out_specs=pl.BlockSpec((1, _E), lambda i: (0, 0)),
        out_shape=jax.ShapeDtypeStruct((1, _E), jnp.float32),
    )(x, W1, b1.reshape(1, _H), W2, b2.reshape(1, _E))
    return out.reshape(_E)
